# baseline (speedup 1.0000x reference)
import functools

import jax
import jax.numpy as jnp
from jax import lax
from jax.experimental import pallas as pl
from jax.experimental.pallas import tpu as pltpu
from jax.experimental.pallas import tpu_sc as plsc

NUM_EMBEDDINGS = 1000000
EMB_DIM = 64
NUM_EDGE_TYPES = 8
LANES = 16
FEAT_CHUNKS = EMB_DIM // LANES


def _shuffle(x, idx):
    dnums = lax.GatherDimensionNumbers(
        offset_dims=(), collapsed_slice_dims=(0,), start_index_map=(0,))
    return lax.gather(
        x, idx[:, None], dnums, slice_sizes=(1,),
        mode=lax.GatherScatterMode.PROMISE_IN_BOUNDS)


@functools.lru_cache(maxsize=None)
def _build(batch: int):
    info = plsc.get_sparse_core_info()
    nc, ns = info.num_cores, info.num_subcores
    nw = nc * ns
    e_per_w = batch // nw
    rows_per_w = 2 * e_per_w
    gather_chunk = 128
    n_gathers = rows_per_w // gather_chunk
    blocks = e_per_w // LANES

    mesh = plsc.VectorSubcoreMesh(core_axis_name="c", subcore_axis_name="s")

    @functools.partial(
        pl.kernel,
        mesh=mesh,
        out_type=jax.ShapeDtypeStruct((batch,), jnp.float32),
        scratch_types=[
            pltpu.VMEM((rows_per_w,), jnp.int32),
            pltpu.VMEM((rows_per_w, EMB_DIM), jnp.float32),
            pltpu.VMEM((e_per_w,), jnp.int32),
            pltpu.VMEM((NUM_EDGE_TYPES, EMB_DIM), jnp.float32),
            pltpu.VMEM((NUM_EDGE_TYPES, EMB_DIM), jnp.float32),
            pltpu.VMEM((e_per_w,), jnp.float32),
            pltpu.SemaphoreType.DMA,
        ],
        compiler_params=pltpu.CompilerParams(use_tc_tiling_on_sc=False),
    )
    def sc_kernel(pairs_hbm, et_hbm, table_hbm, t_hbm, d_hbm, out_hbm,
                  idx_v, rows_v, et_v, t_v, d_v, scores_v, sem):
        wid = lax.axis_index("s") * nc + lax.axis_index("c")
        ebase = wid * e_per_w

        pltpu.sync_copy(pairs_hbm.at[pl.ds(ebase * 2, rows_per_w)], idx_v)
        pltpu.sync_copy(et_hbm.at[pl.ds(ebase, e_per_w)], et_v)
        pltpu.sync_copy(t_hbm, t_v)
        pltpu.sync_copy(d_hbm, d_v)

        copies = []
        for g in range(n_gathers):
            sl = pl.ds(g * gather_chunk, gather_chunk)
            copies.append(
                pltpu.async_copy(table_hbm.at[idx_v.at[sl]], rows_v.at[sl], sem))
        for c in copies:
            c.wait()

        lane_iota = lax.iota(jnp.int32, LANES)
        perms = [lane_iota ^ s for s in (8, 4, 2, 1)]

        def block_body(b, carry):
            et_vec = et_v[pl.ds(b * LANES, LANES)]
            score_vec = jnp.zeros((LANES,), jnp.float32)
            for j in range(LANES):
                e = b * LANES + j
                et = et_vec[j]
                acc = jnp.zeros((LANES,), jnp.float32)
                for c in range(FEAT_CHUNKS):
                    fsl = pl.ds(c * LANES, LANES)
                    src = rows_v[2 * e, fsl]
                    dst = rows_v[2 * e + 1, fsl]
                    tt = t_v[et, fsl]
                    dd = d_v[et, fsl]
                    acc = acc + (src + tt) * (dst * dd)
                for p in perms:
                    acc = acc + _shuffle(acc, p)
                score_vec = jnp.where(lane_iota == j, acc, score_vec)
            scores_v[pl.ds(b * LANES, LANES)] = score_vec
            return carry

        lax.fori_loop(0, blocks, block_body, 0)

        pltpu.sync_copy(scores_v, out_hbm.at[pl.ds(ebase, e_per_w)])

    return sc_kernel


def kernel(src_dst_pairs, condensed_edge_types, table, src_translation, dst_diag):
    batch = condensed_edge_types.shape[0]
    fn = _build(batch)
    return fn(
        jnp.asarray(src_dst_pairs, jnp.int32),
        jnp.asarray(condensed_edge_types, jnp.int32),
        table,
        src_translation,
        dst_diag,
    )


# speedup vs baseline: 1.6652x; 1.6652x over previous
import functools

import jax
import jax.numpy as jnp
from jax import lax
from jax.experimental import pallas as pl
from jax.experimental.pallas import tpu as pltpu
from jax.experimental.pallas import tpu_sc as plsc

NUM_EMBEDDINGS = 1000000
EMB_DIM = 64
NUM_EDGE_TYPES = 8
LANES = 16
FEAT_CHUNKS = EMB_DIM // LANES
CHUNK_ROWS = 256


def _shuffle(x, idx):
    dnums = lax.GatherDimensionNumbers(
        offset_dims=(), collapsed_slice_dims=(0,), start_index_map=(0,))
    return lax.gather(
        x, idx[:, None], dnums, slice_sizes=(1,),
        mode=lax.GatherScatterMode.PROMISE_IN_BOUNDS)


@functools.lru_cache(maxsize=None)
def _build(batch: int):
    info = plsc.get_sparse_core_info()
    nc, ns = info.num_cores, info.num_subcores
    nw = nc * ns
    e_per_w = batch // nw
    rows_per_w = 2 * e_per_w
    n_chunks = rows_per_w // CHUNK_ROWS
    chunk_edges = CHUNK_ROWS // 2

    mesh = plsc.VectorSubcoreMesh(core_axis_name="c", subcore_axis_name="s")

    @functools.partial(
        pl.kernel,
        mesh=mesh,
        out_type=jax.ShapeDtypeStruct((batch,), jnp.float32),
        scratch_types=[
            pltpu.VMEM((rows_per_w,), jnp.int32),
            pltpu.VMEM((CHUNK_ROWS, EMB_DIM), jnp.float32),
            pltpu.VMEM((CHUNK_ROWS, EMB_DIM), jnp.float32),
            pltpu.VMEM((e_per_w,), jnp.int32),
            pltpu.VMEM((NUM_EDGE_TYPES, EMB_DIM), jnp.float32),
            pltpu.VMEM((NUM_EDGE_TYPES, EMB_DIM), jnp.float32),
            pltpu.VMEM((e_per_w,), jnp.float32),
            pltpu.SemaphoreType.DMA,
            pltpu.SemaphoreType.DMA,
        ],
    )
    def sc_kernel(pairs_hbm, et_hbm, table_hbm, t_hbm, d_hbm, out_hbm,
                  idx_v, rows_a, rows_b, et_v, t_v, d_v, scores_v,
                  sem_a, sem_b):
        wid = lax.axis_index("s") * nc + lax.axis_index("c")
        ebase = wid * e_per_w

        pltpu.sync_copy(pairs_hbm.at[pl.ds(ebase * 2, rows_per_w)], idx_v)
        pltpu.sync_copy(et_hbm.at[pl.ds(ebase, e_per_w)], et_v)
        pltpu.sync_copy(t_hbm, t_v)
        pltpu.sync_copy(d_hbm, d_v)

        def fire_chunk(chunk, rows_ref, sem):
            def body(b, carry):
                ids = idx_v[pl.ds(chunk * CHUNK_ROWS + b * LANES, LANES)]
                for j in range(LANES):
                    rid = ids[j]
                    pltpu.async_copy(
                        table_hbm.at[pl.ds(rid, 1), :],
                        rows_ref.at[pl.ds(b * LANES + j, 1), :],
                        sem)
                return carry
            lax.fori_loop(0, CHUNK_ROWS // LANES, body, 0)

        def wait_chunk(rows_ref, sem):
            pltpu.make_async_copy(
                table_hbm.at[pl.ds(0, CHUNK_ROWS), :], rows_ref, sem).wait()

        lane_iota = lax.iota(jnp.int32, LANES)
        perms = [lane_iota ^ s for s in (8, 4, 2, 1)]

        def compute_chunk(chunk, rows_ref):
            def block_body(b, carry):
                e0 = chunk * chunk_edges + b * LANES
                et_vec = et_v[pl.ds(e0, LANES)]
                score_vec = jnp.zeros((LANES,), jnp.float32)
                for j in range(LANES):
                    le = b * LANES + j
                    et = et_vec[j]
                    acc = jnp.zeros((LANES,), jnp.float32)
                    for c in range(FEAT_CHUNKS):
                        fsl = pl.ds(c * LANES, LANES)
                        src = rows_ref[2 * le, fsl]
                        dst = rows_ref[2 * le + 1, fsl]
                        tt = t_v[et, fsl]
                        dd = d_v[et, fsl]
                        acc = acc + (src + tt) * (dst * dd)
                    for p in perms:
                        acc = acc + _shuffle(acc, p)
                    score_vec = jnp.where(lane_iota == j, acc, score_vec)
                scores_v[pl.ds(e0, LANES)] = score_vec
                return carry
            lax.fori_loop(0, chunk_edges // LANES, block_body, 0)

        bufs = (rows_a, rows_b)
        sems = (sem_a, sem_b)
        fire_chunk(0, bufs[0], sems[0])
        for chunk in range(n_chunks):
            if chunk + 1 < n_chunks:
                fire_chunk(chunk + 1, bufs[(chunk + 1) % 2], sems[(chunk + 1) % 2])
            wait_chunk(bufs[chunk % 2], sems[chunk % 2])
            compute_chunk(chunk, bufs[chunk % 2])

        pltpu.sync_copy(scores_v, out_hbm.at[pl.ds(ebase, e_per_w)])

    return sc_kernel


def kernel(src_dst_pairs, condensed_edge_types, table, src_translation, dst_diag):
    batch = condensed_edge_types.shape[0]
    fn = _build(batch)
    return fn(
        jnp.asarray(src_dst_pairs, jnp.int32),
        jnp.asarray(condensed_edge_types, jnp.int32),
        table,
        src_translation,
        dst_diag,
    )
